# baseline (speedup 1.0000x reference)
import jax
import jax.numpy as jnp
from jax import lax
from jax.experimental import pallas as pl
from jax.experimental.pallas import tpu as pltpu
from jax.experimental.pallas import tpu_sc as plsc

B = 100000
NLEV = 4
D = 32
DOUT = NLEV * D
C = 224
NCHUNK = 16
SPAN = C * NCHUNK
NW = 32
NS = 16


def _sc_body(idxT, t0, t1, t2, t3, out,
             iv0, iv1, iv2, iv3, r0, r1, r2, r3, spc,
             isem, gsem, bsem, wsem):
    ivs = (iv0, iv1, iv2, iv3)
    rows = (r0, r1, r2, r3)
    tables = (t0, t1, t2, t3)
    sid = lax.axis_index("s")
    wid = sid * 2 + lax.axis_index("c")
    base = pl.multiple_of(jnp.minimum(wid * SPAN, B - SPAN), 8)
    ih = [pltpu.async_copy(idxT.at[lvl, pl.ds(base, SPAN)],
                           ivs[lvl], isem)
          for lvl in range(NLEV)]
    for h in ih:
        h.wait()
    bigh = [None, None]
    for it in range(NCHUNK):
        bo = (it % 2) * C
        gh = [pltpu.async_copy(
                  tables[lvl].at[ivs[lvl].at[pl.ds(it * C, C)]],
                  rows[lvl], gsem)
              for lvl in range(NLEV)]
        if bigh[it % 2] is not None:
            bigh[it % 2].wait()
        bh = []
        for lvl in range(NLEV):
            gh[lvl].wait()
            bh.append(pltpu.async_copy(
                rows[lvl],
                spc.at[sid, pl.ds(bo, C), pl.ds(lvl * D, D)],
                bsem))
        for h in bh:
            h.wait()
        bigh[it % 2] = pltpu.async_copy(
            spc.at[sid, pl.ds(bo, C), :],
            out.at[pl.ds(base + it * C, C), :],
            wsem)
    for h in bigh:
        h.wait()


def kernel(code_levels, table_0, table_1, table_2, table_3):
    idxT = code_levels.T
    mesh = plsc.VectorSubcoreMesh(core_axis_name="c", subcore_axis_name="s")
    run = pl.kernel(
        _sc_body,
        out_type=jax.ShapeDtypeStruct((B, DOUT), jnp.float32),
        mesh=mesh,
        compiler_params=pltpu.CompilerParams(use_tc_tiling_on_sc=False),
        scratch_types=(
            [pltpu.VMEM((SPAN,), jnp.int32)] * NLEV
            + [pltpu.VMEM((C, D), jnp.float32)] * NLEV
            + [pltpu.VMEM_SHARED((NS, 2 * C, DOUT), jnp.float32)]
            + [pltpu.SemaphoreType.DMA] * 4
        ),
    )
    return run(idxT, table_0, table_1, table_2, table_3)


# speedup vs baseline: 1.0140x; 1.0140x over previous
import jax
import jax.numpy as jnp
from jax import lax
from jax.experimental import pallas as pl
from jax.experimental.pallas import tpu as pltpu
from jax.experimental.pallas import tpu_sc as plsc

B = 100000
NLEV = 4
D = 32
DOUT = NLEV * D
C = 640
NW = 32
NCHUNK = 5


def _sc_body(idxT, t0, t1, t2, t3, out,
             iv0, iv1, iv2, iv3, r0, r1, r2, r3, isem, gsem, wsem):
    ivs = (iv0, iv1, iv2, iv3)
    rows = (r0, r1, r2, r3)
    tables = (t0, t1, t2, t3)
    wid = lax.axis_index("s") * 2 + lax.axis_index("c")
    wh = [None] * NLEV
    for it in range(NCHUNK):
        i = wid + it * NW
        base = pl.multiple_of(jnp.minimum(i * C, B - C), 8)
        ih = [pltpu.async_copy(idxT.at[lvl, pl.ds(base, C)],
                               ivs[lvl], isem)
              for lvl in range(NLEV)]
        gh = []
        for lvl in range(NLEV):
            ih[lvl].wait()
            if wh[lvl] is not None:
                wh[lvl].wait()
            gh.append(pltpu.async_copy(
                tables[lvl].at[ivs[lvl]], rows[lvl], gsem))
        for lvl in range(NLEV):
            gh[lvl].wait()
            wh[lvl] = pltpu.async_copy(
                rows[lvl],
                out.at[pl.ds(base, C), pl.ds(lvl * D, D)],
                wsem)
    for lvl in range(NLEV):
        wh[lvl].wait()


def kernel(code_levels, table_0, table_1, table_2, table_3):
    idxT = code_levels.T
    mesh = plsc.VectorSubcoreMesh(core_axis_name="c", subcore_axis_name="s")
    run = pl.kernel(
        _sc_body,
        out_type=jax.ShapeDtypeStruct((B, DOUT), jnp.float32),
        mesh=mesh,
        compiler_params=pltpu.CompilerParams(use_tc_tiling_on_sc=False),
        scratch_types=(
            [pltpu.VMEM((C,), jnp.int32)] * NLEV
            + [pltpu.VMEM((C, D), jnp.float32)] * NLEV
            + [pltpu.SemaphoreType.DMA] * 3
        ),
    )
    return run(idxT, table_0, table_1, table_2, table_3)
